# baseline (device time: 85934 ns/iter reference)
import jax
import jax.numpy as jnp
from jax import lax
from jax.experimental import pallas as pl
from jax.experimental.pallas import tpu as pltpu

N_DEV = 4
N_SUB = 4


def kernel(A, B):
    m, _ = A.shape
    _, n = B.shape
    ch = m // N_DEV
    nh = n // 2
    sq = nh // N_SUB

    n_msg = (N_DEV - 1) * N_SUB
    n_out = 2 * N_SUB * N_DEV

    def body(a_ref, b_ref, out_ref, af32, bf32, a16, b16, acc,
             rs_send_r, rs_recv_r, ag_send_r, ag_recv_r,
             rs_send_l, rs_recv_l, ag_send_l, ag_recv_l,
             rs_ss_r, rs_rs_r, ag_ss_r, ag_rs_r,
             rs_ss_l, rs_rs_l, ag_ss_l, ag_rs_l,
             a_sems, b_sems, out_sems):
        my = lax.axis_index("i")
        left = lax.rem(my + N_DEV - 1, N_DEV)
        right = lax.rem(my + 1, N_DEV)

        def rows(c):
            return pl.ds(lax.rem(c + 2 * N_DEV, N_DEV) * ch, ch)

        D = (
            dict(rs_send=rs_send_r, rs_recv=rs_recv_r, ag_send=ag_send_r,
                 ag_recv=ag_recv_r, rs_ss=rs_ss_r, rs_rs=rs_rs_r,
                 ag_ss=ag_ss_r, ag_rs=ag_rs_r, nbr=right, col0=nh, sgn=1),
            dict(rs_send=rs_send_l, rs_recv=rs_recv_l, ag_send=ag_send_l,
                 ag_recv=ag_recv_l, rs_ss=rs_ss_l, rs_rs=rs_rs_l,
                 ag_ss=ag_ss_l, ag_rs=ag_rs_l, nbr=left, col0=0, sgn=-1),
        )

        def cols(d, u):
            return slice(d["col0"] + u * sq, d["col0"] + (u + 1) * sq)

        ACC_SLOT = {1: 0, -1: 1, 2: 2}
        RS_SLOT = {(1, 0): 1, (-1, 0): 0, (1, 1): 2, (-1, 1): 2,
                   (1, 2): 0, (-1, 2): 1}

        A_IDX = {0: 0, 1: 1, -1: 2, 2: 3}
        in_cp = {}

        def issue_inputs():
            for u in range(N_SUB):
                for di, d in enumerate(D):
                    cp = pltpu.make_async_copy(
                        b_ref.at[:, cols(d, u)], bf32.at[:, cols(d, u)],
                        b_sems.at[u * 2 + di],
                    )
                    cp.start()
                    in_cp[("b", d["sgn"], u)] = cp
            for dc, j in A_IDX.items():
                cp = pltpu.make_async_copy(
                    a_ref.at[rows(my + dc), :], af32.at[rows(my + dc), :],
                    a_sems.at[j],
                )
                cp.start()
                in_cp[("a", dc)] = cp

        def dot_chunk(dc):
            c = my + dc
            in_cp[("a", dc)].wait()
            a16[rows(c), :] = af32[rows(c), :].astype(jnp.bfloat16)
            acc[ACC_SLOT[dc]] = jnp.dot(
                a16[rows(c), :], b16[...],
                preferred_element_type=jnp.float32,
            )

        def rs_desc(d, s, u):
            i = s * N_SUB + u
            return pltpu.make_async_remote_copy(
                src_ref=d["rs_send"].at[i], dst_ref=d["rs_recv"].at[i],
                send_sem=d["rs_ss"].at[i], recv_sem=d["rs_rs"].at[i],
                device_id=(d["nbr"],), device_id_type=pl.DeviceIdType.MESH,
            )

        def ag_desc(d, t, u):
            i = t * N_SUB + u
            src = d["ag_send"].at[u] if t == 0 else d["ag_recv"].at[i - N_SUB]
            return pltpu.make_async_remote_copy(
                src_ref=src, dst_ref=d["ag_recv"].at[i],
                send_sem=d["ag_ss"].at[i], recv_sem=d["ag_rs"].at[i],
                device_id=(d["nbr"],), device_id_type=pl.DeviceIdType.MESH,
            )

        out_cps = []

        def store_out(src_slice, rc, d, u):
            cp = pltpu.make_async_copy(
                src_slice,
                out_ref.at[rows(rc), cols(d, u)],
                out_sems.at[len(out_cps)],
            )
            cp.start()
            out_cps.append(cp)

        issue_inputs()

        barrier_sem = pltpu.get_barrier_semaphore()
        for nbr in (left, right):
            pl.semaphore_signal(
                barrier_sem, inc=1,
                device_id=(nbr,), device_id_type=pl.DeviceIdType.MESH,
            )
        pl.semaphore_wait(barrier_sem, 2)

        for u in range(N_SUB):
            for di, d in enumerate(D):
                in_cp[("b", d["sgn"], u)].wait()
                b16[:, cols(d, u)] = bf32[:, cols(d, u)].astype(jnp.bfloat16)
                if u == 0 and di == 0:
                    in_cp[("a", 0)].wait()
                    a16[rows(my), :] = af32[rows(my), :].astype(jnp.bfloat16)
                d["rs_send"][u] = jnp.dot(
                    a16[rows(my), :], b16[:, cols(d, u)],
                    preferred_element_type=jnp.float32,
                ).astype(jnp.bfloat16)
                rs_desc(d, 0, u).start()

        dot_chunk(1)
        dot_chunk(-1)

        for s in range(N_DEV - 1):
            if s == 1:
                dot_chunk(2)
            for u in range(N_SUB):
                for d in D:
                    rs_desc(d, s, u).wait_recv()
                    slot = RS_SLOT[(d["sgn"], s)]
                    acc_val = (
                        acc[slot, :, cols(d, u)]
                        + d["rs_recv"][s * N_SUB + u].astype(jnp.float32)
                    )
                    if s < N_DEV - 2:
                        d["rs_send"][(s + 1) * N_SUB + u] = (
                            acc_val.astype(jnp.bfloat16)
                        )
                        rs_desc(d, s + 1, u).start()
                    else:
                        d["ag_send"][u] = acc_val.astype(jnp.bfloat16)
                        ag_desc(d, 0, u).start()
                        store_out(d["ag_send"].at[u], my + d["sgn"], d, u)

        for t in range(N_DEV - 1):
            for u in range(N_SUB):
                for d in D:
                    i = t * N_SUB + u
                    ag_desc(d, t, u).wait_recv()
                    if t < N_DEV - 2:
                        ag_desc(d, t + 1, u).start()
                    store_out(d["ag_recv"].at[i], my - d["sgn"] * t, d, u)

        for d in D:
            for s in range(N_DEV - 1):
                for u in range(N_SUB):
                    rs_desc(d, s, u).wait_send()
                    ag_desc(d, s, u).wait_send()
        for cp in out_cps:
            cp.wait()

    return pl.pallas_call(
        body,
        out_shape=jax.ShapeDtypeStruct((m, n), jnp.bfloat16),
        in_specs=[
            pl.BlockSpec(memory_space=pl.ANY),
            pl.BlockSpec(memory_space=pl.ANY),
        ],
        out_specs=pl.BlockSpec(memory_space=pl.ANY),
        scratch_shapes=[
            pltpu.VMEM(A.shape, jnp.float32),
            pltpu.VMEM(B.shape, jnp.float32),
            pltpu.VMEM(A.shape, jnp.bfloat16),
            pltpu.VMEM(B.shape, jnp.bfloat16),
            pltpu.VMEM((3, ch, n), jnp.float32),
            pltpu.VMEM((n_msg, ch, sq), jnp.bfloat16),
            pltpu.VMEM((n_msg, ch, sq), jnp.bfloat16),
            pltpu.VMEM((N_SUB, ch, sq), jnp.bfloat16),
            pltpu.VMEM((n_msg, ch, sq), jnp.bfloat16),
            pltpu.VMEM((n_msg, ch, sq), jnp.bfloat16),
            pltpu.VMEM((n_msg, ch, sq), jnp.bfloat16),
            pltpu.VMEM((N_SUB, ch, sq), jnp.bfloat16),
            pltpu.VMEM((n_msg, ch, sq), jnp.bfloat16),
            pltpu.SemaphoreType.DMA((n_msg,)),
            pltpu.SemaphoreType.DMA((n_msg,)),
            pltpu.SemaphoreType.DMA((n_msg,)),
            pltpu.SemaphoreType.DMA((n_msg,)),
            pltpu.SemaphoreType.DMA((n_msg,)),
            pltpu.SemaphoreType.DMA((n_msg,)),
            pltpu.SemaphoreType.DMA((n_msg,)),
            pltpu.SemaphoreType.DMA((n_msg,)),
            pltpu.SemaphoreType.DMA((4,)),
            pltpu.SemaphoreType.DMA((2 * N_SUB,)),
            pltpu.SemaphoreType.DMA((n_out,)),
        ],
        compiler_params=pltpu.CompilerParams(
            collective_id=0, vmem_limit_bytes=96 * 1024 * 1024
        ),
    )(A, B)


# device time: 84019 ns/iter; 1.0228x vs baseline; 1.0228x over previous
import jax
import jax.numpy as jnp
from jax import lax
from jax.experimental import pallas as pl
from jax.experimental.pallas import tpu as pltpu

N_DEV = 4
N_SUB = 2


def kernel(A, B):
    m, _ = A.shape
    _, n = B.shape
    ch = m // N_DEV
    nh = n // 2
    sq = nh // N_SUB

    n_msg = (N_DEV - 1) * N_SUB
    n_out = 2 * N_SUB * N_DEV

    def body(a_ref, b_ref, out_ref, af32, bf32, a16, b16, acc,
             rs_send_r, rs_recv_r, ag_send_r, ag_recv_r,
             rs_send_l, rs_recv_l, ag_send_l, ag_recv_l,
             rs_ss_r, rs_rs_r, ag_ss_r, ag_rs_r,
             rs_ss_l, rs_rs_l, ag_ss_l, ag_rs_l,
             a_sems, b_sems, out_sems):
        my = lax.axis_index("i")
        left = lax.rem(my + N_DEV - 1, N_DEV)
        right = lax.rem(my + 1, N_DEV)

        def rows(c):
            return pl.ds(lax.rem(c + 2 * N_DEV, N_DEV) * ch, ch)

        D = (
            dict(rs_send=rs_send_r, rs_recv=rs_recv_r, ag_send=ag_send_r,
                 ag_recv=ag_recv_r, rs_ss=rs_ss_r, rs_rs=rs_rs_r,
                 ag_ss=ag_ss_r, ag_rs=ag_rs_r, nbr=right, col0=nh, sgn=1),
            dict(rs_send=rs_send_l, rs_recv=rs_recv_l, ag_send=ag_send_l,
                 ag_recv=ag_recv_l, rs_ss=rs_ss_l, rs_rs=rs_rs_l,
                 ag_ss=ag_ss_l, ag_rs=ag_rs_l, nbr=left, col0=0, sgn=-1),
        )

        def cols(d, u):
            return slice(d["col0"] + u * sq, d["col0"] + (u + 1) * sq)

        ACC_SLOT = {1: 0, -1: 1, 2: 2}
        RS_SLOT = {(1, 0): 1, (-1, 0): 0, (1, 1): 2, (-1, 1): 2,
                   (1, 2): 0, (-1, 2): 1}

        A_IDX = {0: 0, 1: 1, -1: 2, 2: 3}
        in_cp = {}

        def issue_b(u, di, d):
            cp = pltpu.make_async_copy(
                b_ref.at[:, cols(d, u)], bf32.at[:, cols(d, u)],
                b_sems.at[u * 2 + di],
            )
            cp.start()
            in_cp[("b", d["sgn"], u)] = cp

        def issue_a(dc):
            cp = pltpu.make_async_copy(
                a_ref.at[rows(my + dc), :], af32.at[rows(my + dc), :],
                a_sems.at[A_IDX[dc]],
            )
            cp.start()
            in_cp[("a", dc)] = cp

        def dot_chunk(dc):
            c = my + dc
            in_cp[("a", dc)].wait()
            a16[rows(c), :] = af32[rows(c), :].astype(jnp.bfloat16)
            acc[ACC_SLOT[dc]] = jnp.dot(
                a16[rows(c), :], b16[...],
                preferred_element_type=jnp.float32,
            )

        def rs_desc(d, s, u):
            i = s * N_SUB + u
            return pltpu.make_async_remote_copy(
                src_ref=d["rs_send"].at[i], dst_ref=d["rs_recv"].at[i],
                send_sem=d["rs_ss"].at[i], recv_sem=d["rs_rs"].at[i],
                device_id=(d["nbr"],), device_id_type=pl.DeviceIdType.MESH,
            )

        def ag_desc(d, t, u):
            i = t * N_SUB + u
            src = d["ag_send"].at[u] if t == 0 else d["ag_recv"].at[i - N_SUB]
            return pltpu.make_async_remote_copy(
                src_ref=src, dst_ref=d["ag_recv"].at[i],
                send_sem=d["ag_ss"].at[i], recv_sem=d["ag_rs"].at[i],
                device_id=(d["nbr"],), device_id_type=pl.DeviceIdType.MESH,
            )

        out_cps = []

        def store_out(src_slice, rc, d, u):
            cp = pltpu.make_async_copy(
                src_slice,
                out_ref.at[rows(rc), cols(d, u)],
                out_sems.at[len(out_cps)],
            )
            cp.start()
            out_cps.append(cp)

        issue_b(0, 0, D[0])
        issue_a(0)

        barrier_sem = pltpu.get_barrier_semaphore()
        for nbr in (left, right):
            pl.semaphore_signal(
                barrier_sem, inc=1,
                device_id=(nbr,), device_id_type=pl.DeviceIdType.MESH,
            )
        pl.semaphore_wait(barrier_sem, 2)

        issue_b(0, 1, D[1])
        for u in range(1, N_SUB):
            for di, d in enumerate(D):
                issue_b(u, di, d)
        for dc in (1, -1, 2):
            issue_a(dc)

        for u in range(N_SUB):
            for di, d in enumerate(D):
                in_cp[("b", d["sgn"], u)].wait()
                b16[:, cols(d, u)] = bf32[:, cols(d, u)].astype(jnp.bfloat16)
                if u == 0 and di == 0:
                    in_cp[("a", 0)].wait()
                    a16[rows(my), :] = af32[rows(my), :].astype(jnp.bfloat16)
                d["rs_send"][u] = jnp.dot(
                    a16[rows(my), :], b16[:, cols(d, u)],
                    preferred_element_type=jnp.float32,
                ).astype(jnp.bfloat16)
                rs_desc(d, 0, u).start()

        dot_chunk(1)
        dot_chunk(-1)

        for s in range(N_DEV - 1):
            if s == 1:
                dot_chunk(2)
            for u in range(N_SUB):
                for d in D:
                    rs_desc(d, s, u).wait_recv()
                    slot = RS_SLOT[(d["sgn"], s)]
                    acc_val = (
                        acc[slot, :, cols(d, u)]
                        + d["rs_recv"][s * N_SUB + u].astype(jnp.float32)
                    )
                    if s < N_DEV - 2:
                        d["rs_send"][(s + 1) * N_SUB + u] = (
                            acc_val.astype(jnp.bfloat16)
                        )
                        rs_desc(d, s + 1, u).start()
                    else:
                        d["ag_send"][u] = acc_val.astype(jnp.bfloat16)
                        ag_desc(d, 0, u).start()
                        store_out(d["ag_send"].at[u], my + d["sgn"], d, u)

        for t in range(N_DEV - 1):
            for u in range(N_SUB):
                for d in D:
                    i = t * N_SUB + u
                    ag_desc(d, t, u).wait_recv()
                    if t < N_DEV - 2:
                        ag_desc(d, t + 1, u).start()
                    store_out(d["ag_recv"].at[i], my - d["sgn"] * t, d, u)

        for d in D:
            for s in range(N_DEV - 1):
                for u in range(N_SUB):
                    rs_desc(d, s, u).wait_send()
                    ag_desc(d, s, u).wait_send()
        for cp in out_cps:
            cp.wait()

    return pl.pallas_call(
        body,
        out_shape=jax.ShapeDtypeStruct((m, n), jnp.bfloat16),
        in_specs=[
            pl.BlockSpec(memory_space=pl.ANY),
            pl.BlockSpec(memory_space=pl.ANY),
        ],
        out_specs=pl.BlockSpec(memory_space=pl.ANY),
        scratch_shapes=[
            pltpu.VMEM(A.shape, jnp.float32),
            pltpu.VMEM(B.shape, jnp.float32),
            pltpu.VMEM(A.shape, jnp.bfloat16),
            pltpu.VMEM(B.shape, jnp.bfloat16),
            pltpu.VMEM((3, ch, n), jnp.float32),
            pltpu.VMEM((n_msg, ch, sq), jnp.bfloat16),
            pltpu.VMEM((n_msg, ch, sq), jnp.bfloat16),
            pltpu.VMEM((N_SUB, ch, sq), jnp.bfloat16),
            pltpu.VMEM((n_msg, ch, sq), jnp.bfloat16),
            pltpu.VMEM((n_msg, ch, sq), jnp.bfloat16),
            pltpu.VMEM((n_msg, ch, sq), jnp.bfloat16),
            pltpu.VMEM((N_SUB, ch, sq), jnp.bfloat16),
            pltpu.VMEM((n_msg, ch, sq), jnp.bfloat16),
            pltpu.SemaphoreType.DMA((n_msg,)),
            pltpu.SemaphoreType.DMA((n_msg,)),
            pltpu.SemaphoreType.DMA((n_msg,)),
            pltpu.SemaphoreType.DMA((n_msg,)),
            pltpu.SemaphoreType.DMA((n_msg,)),
            pltpu.SemaphoreType.DMA((n_msg,)),
            pltpu.SemaphoreType.DMA((n_msg,)),
            pltpu.SemaphoreType.DMA((n_msg,)),
            pltpu.SemaphoreType.DMA((4,)),
            pltpu.SemaphoreType.DMA((2 * N_SUB,)),
            pltpu.SemaphoreType.DMA((n_out,)),
        ],
        compiler_params=pltpu.CompilerParams(
            collective_id=0, vmem_limit_bytes=96 * 1024 * 1024
        ),
    )(A, B)


# device time: 84009 ns/iter; 1.0229x vs baseline; 1.0001x over previous
import jax
import jax.numpy as jnp
from jax import lax
from jax.experimental import pallas as pl
from jax.experimental.pallas import tpu as pltpu

N_DEV = 4
N_SUB = 2


def kernel(A, B):
    m, _ = A.shape
    _, n = B.shape
    ch = m // N_DEV
    nh = n // 2
    sq = nh // N_SUB

    n_msg = (N_DEV - 1) * N_SUB
    n_out = 2 * N_SUB * N_DEV

    def body(a_ref, b_ref, out_ref, af32, bf32, a16, b16, acc,
             rs_send_r, rs_recv_r, ag_send_r, ag_recv_r,
             rs_send_l, rs_recv_l, ag_send_l, ag_recv_l,
             rs_ss_r, rs_rs_r, ag_ss_r, ag_rs_r,
             rs_ss_l, rs_rs_l, ag_ss_l, ag_rs_l,
             a_sems, b_sems, out_sems):
        my = lax.axis_index("i")
        left = lax.rem(my + N_DEV - 1, N_DEV)
        right = lax.rem(my + 1, N_DEV)

        def rows(c):
            return pl.ds(lax.rem(c + 2 * N_DEV, N_DEV) * ch, ch)

        D = (
            dict(rs_send=rs_send_r, rs_recv=rs_recv_r, ag_send=ag_send_r,
                 ag_recv=ag_recv_r, rs_ss=rs_ss_r, rs_rs=rs_rs_r,
                 ag_ss=ag_ss_r, ag_rs=ag_rs_r, nbr=right, col0=nh, sgn=1),
            dict(rs_send=rs_send_l, rs_recv=rs_recv_l, ag_send=ag_send_l,
                 ag_recv=ag_recv_l, rs_ss=rs_ss_l, rs_rs=rs_rs_l,
                 ag_ss=ag_ss_l, ag_rs=ag_rs_l, nbr=left, col0=0, sgn=-1),
        )

        def cols(d, u):
            return slice(d["col0"] + u * sq, d["col0"] + (u + 1) * sq)

        ACC_SLOT = {1: 0, -1: 1, 2: 2}
        RS_SLOT = {(1, 0): 1, (-1, 0): 0, (1, 1): 2, (-1, 1): 2,
                   (1, 2): 0, (-1, 2): 1}

        A_IDX = {0: 0, 1: 1, -1: 2, 2: 3}
        in_cp = {}

        def issue_b(u, di, d):
            cp = pltpu.make_async_copy(
                b_ref.at[:, cols(d, u)], bf32.at[:, cols(d, u)],
                b_sems.at[u * 2 + di],
            )
            cp.start()
            in_cp[("b", d["sgn"], u)] = cp

        def issue_a(dc):
            cp = pltpu.make_async_copy(
                a_ref.at[rows(my + dc), :], af32.at[rows(my + dc), :],
                a_sems.at[A_IDX[dc]],
            )
            cp.start()
            in_cp[("a", dc)] = cp

        def dot_chunk(dc):
            c = my + dc
            in_cp[("a", dc)].wait()
            a16[rows(c), :] = af32[rows(c), :].astype(jnp.bfloat16)
            acc[ACC_SLOT[dc]] = jnp.dot(
                a16[rows(c), :], b16[...],
                preferred_element_type=jnp.float32,
            ).astype(jnp.bfloat16)

        def rs_desc(d, s, u):
            i = s * N_SUB + u
            return pltpu.make_async_remote_copy(
                src_ref=d["rs_send"].at[i], dst_ref=d["rs_recv"].at[i],
                send_sem=d["rs_ss"].at[i], recv_sem=d["rs_rs"].at[i],
                device_id=(d["nbr"],), device_id_type=pl.DeviceIdType.MESH,
            )

        def ag_desc(d, t, u):
            i = t * N_SUB + u
            src = d["ag_send"].at[u] if t == 0 else d["ag_recv"].at[i - N_SUB]
            return pltpu.make_async_remote_copy(
                src_ref=src, dst_ref=d["ag_recv"].at[i],
                send_sem=d["ag_ss"].at[i], recv_sem=d["ag_rs"].at[i],
                device_id=(d["nbr"],), device_id_type=pl.DeviceIdType.MESH,
            )

        out_cps = []

        def store_out(src_slice, rc, d, u):
            cp = pltpu.make_async_copy(
                src_slice,
                out_ref.at[rows(rc), cols(d, u)],
                out_sems.at[len(out_cps)],
            )
            cp.start()
            out_cps.append(cp)

        issue_b(0, 0, D[0])
        issue_a(0)

        barrier_sem = pltpu.get_barrier_semaphore()
        for nbr in (left, right):
            pl.semaphore_signal(
                barrier_sem, inc=1,
                device_id=(nbr,), device_id_type=pl.DeviceIdType.MESH,
            )
        pl.semaphore_wait(barrier_sem, 2)

        issue_b(0, 1, D[1])
        for u in range(1, N_SUB):
            for di, d in enumerate(D):
                issue_b(u, di, d)
        for dc in (1, -1, 2):
            issue_a(dc)

        for u in range(N_SUB):
            for di, d in enumerate(D):
                in_cp[("b", d["sgn"], u)].wait()
                b16[:, cols(d, u)] = bf32[:, cols(d, u)].astype(jnp.bfloat16)
                if u == 0 and di == 0:
                    in_cp[("a", 0)].wait()
                    a16[rows(my), :] = af32[rows(my), :].astype(jnp.bfloat16)
                d["rs_send"][u] = jnp.dot(
                    a16[rows(my), :], b16[:, cols(d, u)],
                    preferred_element_type=jnp.float32,
                ).astype(jnp.bfloat16)
                rs_desc(d, 0, u).start()

        dot_chunk(1)
        dot_chunk(-1)

        for s in range(N_DEV - 1):
            if s == 1:
                dot_chunk(2)
            for u in range(N_SUB):
                for d in D:
                    rs_desc(d, s, u).wait_recv()
                    slot = RS_SLOT[(d["sgn"], s)]
                    acc_val = (
                        acc[slot, :, cols(d, u)]
                        + d["rs_recv"][s * N_SUB + u]
                    )
                    if s < N_DEV - 2:
                        d["rs_send"][(s + 1) * N_SUB + u] = acc_val
                        rs_desc(d, s + 1, u).start()
                    else:
                        d["ag_send"][u] = acc_val
                        ag_desc(d, 0, u).start()
                        store_out(d["ag_send"].at[u], my + d["sgn"], d, u)

        for t in range(N_DEV - 1):
            for u in range(N_SUB):
                for d in D:
                    i = t * N_SUB + u
                    ag_desc(d, t, u).wait_recv()
                    if t < N_DEV - 2:
                        ag_desc(d, t + 1, u).start()
                    store_out(d["ag_recv"].at[i], my - d["sgn"] * t, d, u)

        for d in D:
            for s in range(N_DEV - 1):
                for u in range(N_SUB):
                    rs_desc(d, s, u).wait_send()
                    ag_desc(d, s, u).wait_send()
        for cp in out_cps:
            cp.wait()

    return pl.pallas_call(
        body,
        out_shape=jax.ShapeDtypeStruct((m, n), jnp.bfloat16),
        in_specs=[
            pl.BlockSpec(memory_space=pl.ANY),
            pl.BlockSpec(memory_space=pl.ANY),
        ],
        out_specs=pl.BlockSpec(memory_space=pl.ANY),
        scratch_shapes=[
            pltpu.VMEM(A.shape, jnp.float32),
            pltpu.VMEM(B.shape, jnp.float32),
            pltpu.VMEM(A.shape, jnp.bfloat16),
            pltpu.VMEM(B.shape, jnp.bfloat16),
            pltpu.VMEM((3, ch, n), jnp.bfloat16),
            pltpu.VMEM((n_msg, ch, sq), jnp.bfloat16),
            pltpu.VMEM((n_msg, ch, sq), jnp.bfloat16),
            pltpu.VMEM((N_SUB, ch, sq), jnp.bfloat16),
            pltpu.VMEM((n_msg, ch, sq), jnp.bfloat16),
            pltpu.VMEM((n_msg, ch, sq), jnp.bfloat16),
            pltpu.VMEM((n_msg, ch, sq), jnp.bfloat16),
            pltpu.VMEM((N_SUB, ch, sq), jnp.bfloat16),
            pltpu.VMEM((n_msg, ch, sq), jnp.bfloat16),
            pltpu.SemaphoreType.DMA((n_msg,)),
            pltpu.SemaphoreType.DMA((n_msg,)),
            pltpu.SemaphoreType.DMA((n_msg,)),
            pltpu.SemaphoreType.DMA((n_msg,)),
            pltpu.SemaphoreType.DMA((n_msg,)),
            pltpu.SemaphoreType.DMA((n_msg,)),
            pltpu.SemaphoreType.DMA((n_msg,)),
            pltpu.SemaphoreType.DMA((n_msg,)),
            pltpu.SemaphoreType.DMA((4,)),
            pltpu.SemaphoreType.DMA((2 * N_SUB,)),
            pltpu.SemaphoreType.DMA((n_out,)),
        ],
        compiler_params=pltpu.CompilerParams(
            collective_id=0, vmem_limit_bytes=96 * 1024 * 1024
        ),
    )(A, B)
